# Initial kernel scaffold; baseline (speedup 1.0000x reference)
import jax
import jax.numpy as jnp
from jax.experimental import pallas as pl


def kernel(pred, target, landmarks):
    raise NotImplementedError("write your pallas kernel here")


# speedup vs baseline: 1.9379x; 1.9379x over previous
import jax
import jax.numpy as jnp
from jax.experimental import pallas as pl
from jax.experimental.pallas import tpu as pltpu

_EYE_S = 32
_PAD = 0.3


def _coords_1d(lo, hi, other_lo, other_hi, size):
    b_lo = jnp.clip(lo, 0.0, size - 1.0)
    b_hi = jnp.clip(hi, 0.0, size - 1.0)
    ob_lo = jnp.clip(other_lo, 0.0, size - 1.0)
    ob_hi = jnp.clip(other_hi, 0.0, size - 1.0)
    degenerate = (b_hi - b_lo < 1.0) | (ob_hi - ob_lo < 1.0)
    n0 = b_lo / (size - 1) * 2.0 - 1.0
    n1 = b_hi / (size - 1) * 2.0 - 1.0
    t = jnp.arange(_EYE_S, dtype=jnp.float32) / (_EYE_S - 1)
    g = n0[:, None] + (n1 - n0)[:, None] * t
    g = jnp.where(degenerate[:, None], 0.0, g)
    return jnp.clip((g + 1.0) * 0.5 * (size - 1), 0.0, size - 1.0)


def _eye_coords(landmarks, lo_idx, hi_idx, H, W):
    pts = landmarks[:, lo_idx:hi_idx, :]
    x_min = pts[:, :, 0].min(axis=1)
    x_max = pts[:, :, 0].max(axis=1)
    y_min = pts[:, :, 1].min(axis=1)
    y_max = pts[:, :, 1].max(axis=1)
    w = x_max - x_min
    h = y_max - y_min
    x1, y1 = x_min - w * _PAD, y_min - h * _PAD
    x2, y2 = x_max + w * _PAD, y_max + h * _PAD
    px = _coords_1d(x1, x2, y1, y2, W)
    py = _coords_1d(y1, y2, x1, x2, H)
    return px, py


def _interp_rows(p_col, H):
    p0 = jnp.floor(p_col)
    w = p_col - p0
    i0 = p0.astype(jnp.int32)
    i1 = jnp.minimum(i0 + 1, H - 1)
    io = jax.lax.broadcasted_iota(jnp.int32, (_EYE_S, H), 1)
    return jnp.where(io == i0, 1.0 - w, 0.0) + jnp.where(io == i1, w, 0.0)


def _interp_cols(p_row, W):
    p0 = jnp.floor(p_row)
    w = p_row - p0
    i0 = p0.astype(jnp.int32)
    i1 = jnp.minimum(i0 + 1, W - 1)
    io = jax.lax.broadcasted_iota(jnp.int32, (W, _EYE_S), 0)
    return jnp.where(io == i0, 1.0 - w, 0.0) + jnp.where(io == i1, w, 0.0)


def _gaze_kernel(pred_ref, target_ref, cc_ref, rc_ref, out_ref):
    C = pred_ref.shape[1]
    H = pred_ref.shape[2]
    W = pred_ref.shape[3]
    ry_l = _interp_rows(rc_ref[0, :, 0:1], H)
    ry_r = _interp_rows(rc_ref[0, :, 1:2], H)
    cx_l = _interp_cols(cc_ref[0, 0:1, :], W)
    cx_r = _interp_cols(cc_ref[0, 1:2, :], W)
    acc = jnp.zeros((_EYE_S, _EYE_S), jnp.float32)
    for c in range(C):
        d = pred_ref[0, c] - target_ref[0, c]
        t_l = jnp.dot(ry_l, d,
                      preferred_element_type=jnp.float32,
                      precision=jax.lax.Precision.HIGHEST)
        t_r = jnp.dot(ry_r, d,
                      preferred_element_type=jnp.float32,
                      precision=jax.lax.Precision.HIGHEST)
        p_l = jnp.dot(t_l, cx_l, preferred_element_type=jnp.float32,
                      precision=jax.lax.Precision.HIGHEST)
        p_r = jnp.dot(t_r, cx_r, preferred_element_type=jnp.float32,
                      precision=jax.lax.Precision.HIGHEST)
        acc = acc + jnp.abs(p_l) + jnp.abs(p_r)
    out_ref[...] = jnp.sum(acc, keepdims=True).reshape(1, 1, 1)


def kernel(pred, target, landmarks):
    B, C, H, W = pred.shape
    S = _EYE_S
    lm = jax.lax.stop_gradient(landmarks)
    px_l, py_l = _eye_coords(lm, 36, 42, H, W)
    px_r, py_r = _eye_coords(lm, 42, 48, H, W)
    ccoords = jnp.stack([px_l, px_r], axis=1)
    rcoords = jnp.stack([py_l, py_r], axis=2)

    out = pl.pallas_call(
        _gaze_kernel,
        grid=(B,),
        in_specs=[
            pl.BlockSpec((1, C, H, W), lambda b: (b, 0, 0, 0)),
            pl.BlockSpec((1, C, H, W), lambda b: (b, 0, 0, 0)),
            pl.BlockSpec((1, 2, S), lambda b: (b, 0, 0)),
            pl.BlockSpec((1, S, 2), lambda b: (b, 0, 0)),
        ],
        out_specs=pl.BlockSpec((1, 1, 1), lambda b: (b, 0, 0)),
        out_shape=jax.ShapeDtypeStruct((B, 1, 1), jnp.float32),
        compiler_params=pltpu.CompilerParams(
            dimension_semantics=("parallel",),
        ),
    )(pred, target, ccoords, rcoords)

    return jnp.sum(out) / jnp.float32(2 * B * C * S * S)


# speedup vs baseline: 3.9643x; 2.0457x over previous
import jax
import jax.numpy as jnp
from jax.experimental import pallas as pl
from jax.experimental.pallas import tpu as pltpu

_EYE_S = 32
_PAD = 0.3


def _coords_1d(lo, hi, other_lo, other_hi, size):
    b_lo = jnp.clip(lo, 0.0, size - 1.0)
    b_hi = jnp.clip(hi, 0.0, size - 1.0)
    ob_lo = jnp.clip(other_lo, 0.0, size - 1.0)
    ob_hi = jnp.clip(other_hi, 0.0, size - 1.0)
    degenerate = (b_hi - b_lo < 1.0) | (ob_hi - ob_lo < 1.0)
    n0 = b_lo / (size - 1) * 2.0 - 1.0
    n1 = b_hi / (size - 1) * 2.0 - 1.0
    t = jnp.arange(_EYE_S, dtype=jnp.float32) / (_EYE_S - 1)
    g = n0[:, None] + (n1 - n0)[:, None] * t
    g = jnp.where(degenerate[:, None], 0.0, g)
    return jnp.clip((g + 1.0) * 0.5 * (size - 1), 0.0, size - 1.0)


def _eye_coords(landmarks, lo_idx, hi_idx, H, W):
    pts = landmarks[:, lo_idx:hi_idx, :]
    x_min = pts[:, :, 0].min(axis=1)
    x_max = pts[:, :, 0].max(axis=1)
    y_min = pts[:, :, 1].min(axis=1)
    y_max = pts[:, :, 1].max(axis=1)
    w = x_max - x_min
    h = y_max - y_min
    x1, y1 = x_min - w * _PAD, y_min - h * _PAD
    x2, y2 = x_max + w * _PAD, y_max + h * _PAD
    px = _coords_1d(x1, x2, y1, y2, W)
    py = _coords_1d(y1, y2, x1, x2, H)
    return px, py


def _interp_rows(p_col, H):
    p0 = jnp.floor(p_col)
    w = p_col - p0
    i0 = p0.astype(jnp.int32)
    i1 = jnp.minimum(i0 + 1, H - 1)
    io = jax.lax.broadcasted_iota(jnp.int32, (_EYE_S, H), 1)
    return jnp.where(io == i0, 1.0 - w, 0.0) + jnp.where(io == i1, w, 0.0)


def _interp_cols(p_row, W):
    p0 = jnp.floor(p_row)
    w = p_row - p0
    i0 = p0.astype(jnp.int32)
    i1 = jnp.minimum(i0 + 1, W - 1)
    io = jax.lax.broadcasted_iota(jnp.int32, (W, _EYE_S), 0)
    return jnp.where(io == i0, 1.0 - w, 0.0) + jnp.where(io == i1, w, 0.0)


def _gaze_kernel(pred_ref, target_ref, cc_ref, rc_ref, out_ref):
    C = pred_ref.shape[1]
    H = pred_ref.shape[2]
    W = pred_ref.shape[3]
    ry_l = _interp_rows(rc_ref[0, :, 0:1], H)
    ry_r = _interp_rows(rc_ref[0, :, 1:2], H)
    cx_l = _interp_cols(cc_ref[0, 0:1, :], W)
    cx_r = _interp_cols(cc_ref[0, 1:2, :], W)
    acc = jnp.zeros((_EYE_S, _EYE_S), jnp.float32)
    for c in range(C):
        d = pred_ref[0, c] - target_ref[0, c]
        t_l = jnp.dot(ry_l, d,
                      preferred_element_type=jnp.float32,
                      precision=jax.lax.Precision.DEFAULT)
        t_r = jnp.dot(ry_r, d,
                      preferred_element_type=jnp.float32,
                      precision=jax.lax.Precision.DEFAULT)
        p_l = jnp.dot(t_l, cx_l, preferred_element_type=jnp.float32,
                      precision=jax.lax.Precision.DEFAULT)
        p_r = jnp.dot(t_r, cx_r, preferred_element_type=jnp.float32,
                      precision=jax.lax.Precision.DEFAULT)
        acc = acc + jnp.abs(p_l) + jnp.abs(p_r)
    out_ref[...] = jnp.sum(acc, keepdims=True).reshape(1, 1, 1)


def kernel(pred, target, landmarks):
    B, C, H, W = pred.shape
    S = _EYE_S
    lm = jax.lax.stop_gradient(landmarks)
    px_l, py_l = _eye_coords(lm, 36, 42, H, W)
    px_r, py_r = _eye_coords(lm, 42, 48, H, W)
    ccoords = jnp.stack([px_l, px_r], axis=1)
    rcoords = jnp.stack([py_l, py_r], axis=2)

    halfB = B // 2
    out = pl.pallas_call(
        _gaze_kernel,
        grid=(2, halfB),
        in_specs=[
            pl.BlockSpec((1, C, H, W), lambda i, j: (i * halfB + j, 0, 0, 0)),
            pl.BlockSpec((1, C, H, W), lambda i, j: (i * halfB + j, 0, 0, 0)),
            pl.BlockSpec((1, 2, S), lambda i, j: (i * halfB + j, 0, 0)),
            pl.BlockSpec((1, S, 2), lambda i, j: (i * halfB + j, 0, 0)),
        ],
        out_specs=pl.BlockSpec((1, 1, 1), lambda i, j: (i * halfB + j, 0, 0)),
        out_shape=jax.ShapeDtypeStruct((B, 1, 1), jnp.float32),
        compiler_params=pltpu.CompilerParams(
            dimension_semantics=("parallel", "arbitrary"),
        ),
    )(pred, target, ccoords, rcoords)

    return jnp.sum(out) / jnp.float32(2 * B * C * S * S)
